# Initial kernel scaffold; baseline (speedup 1.0000x reference)
import jax
import jax.numpy as jnp
from jax.experimental import pallas as pl


def kernel(x, edge_index, edge_attr, W_e1, b_e1, W_e2, b_e2, g_e, be_e, W_n1, b_n1, W_n2, b_n2, g_n, be_n):
    raise NotImplementedError("write your pallas kernel here")


# speedup vs baseline: 2.7483x; 2.7483x over previous
import functools

import jax
import jax.numpy as jnp
from jax import lax
from jax.experimental import pallas as pl
from jax.experimental.pallas import tpu as pltpu
from jax.experimental.pallas import tpu_sc as plsc

N_NODES = 10000
N_EDGES = 320000
D = 128
E_DIM = 16

NW = 32
EPW = N_EDGES // NW
CH = 80
NCH = EPW // CH
RPS = 624

_EPS = 1e-5


def _ln(h, gamma, beta):
    mu = jnp.mean(h, axis=-1, keepdims=True)
    var = jnp.mean((h - mu) ** 2, axis=-1, keepdims=True)
    return (h - mu) * lax.rsqrt(var + _EPS) * gamma + beta


def _premix_body(x_ref, w1s_ref, w1r_ref, be1_ref, a_ref, b_ref):
    x = x_ref[...]
    a_ref[...] = jnp.dot(x, w1s_ref[...], preferred_element_type=jnp.float32)
    b_ref[...] = jnp.dot(x, w1r_ref[...], preferred_element_type=jnp.float32) + be1_ref[...]


def _premix(x, w1s, w1r, b_e1):
    blk = 2000
    grid = N_NODES // blk
    return pl.pallas_call(
        _premix_body,
        grid=(grid,),
        in_specs=[
            pl.BlockSpec((blk, D), lambda i: (i, 0)),
            pl.BlockSpec((D, D), lambda i: (0, 0)),
            pl.BlockSpec((D, D), lambda i: (0, 0)),
            pl.BlockSpec((1, D), lambda i: (0, 0)),
        ],
        out_specs=[
            pl.BlockSpec((blk, D), lambda i: (i, 0)),
            pl.BlockSpec((blk, D), lambda i: (i, 0)),
        ],
        out_shape=[
            jax.ShapeDtypeStruct((N_NODES, D), jnp.float32),
            jax.ShapeDtypeStruct((N_NODES, D), jnp.float32),
        ],
    )(x, w1s, w1r, b_e1.reshape(1, D))


def _gather_body(send_ref, recv_ref, a_ref, b_ref, gs_ref, gr_ref,
                 idx_s, idx_r, bufa, bufb, sem_a, sem_b):
    cid = lax.axis_index("c")
    sid = lax.axis_index("s")
    wid = sid * 2 + cid

    def body(i, carry):
        base = wid * EPW + i * CH
        pltpu.sync_copy(send_ref.at[pl.ds(base, CH)], idx_s)
        pltpu.sync_copy(recv_ref.at[pl.ds(base, CH)], idx_r)
        ca = pltpu.async_copy(a_ref.at[idx_s], bufa, sem_a)
        cb = pltpu.async_copy(b_ref.at[idx_r], bufb, sem_b)
        ca.wait()
        cb.wait()
        pltpu.sync_copy(bufa, gs_ref.at[pl.ds(base, CH)])
        pltpu.sync_copy(bufb, gr_ref.at[pl.ds(base, CH)])
        return carry

    lax.fori_loop(0, NCH, body, 0)


def _edge_gather(senders, receivers, a, b):
    mesh = plsc.VectorSubcoreMesh(core_axis_name="c", subcore_axis_name="s")
    return pl.kernel(
        _gather_body,
        out_type=[
            jax.ShapeDtypeStruct((N_EDGES, D), jnp.float32),
            jax.ShapeDtypeStruct((N_EDGES, D), jnp.float32),
        ],
        mesh=mesh,
        scratch_types=[
            pltpu.VMEM((CH,), jnp.int32),
            pltpu.VMEM((CH,), jnp.int32),
            pltpu.VMEM((CH, D), jnp.float32),
            pltpu.VMEM((CH, D), jnp.float32),
            pltpu.SemaphoreType.DMA,
            pltpu.SemaphoreType.DMA,
        ],
    )(senders, receivers, a, b)


def _edge_mlp_body(gs_ref, gr_ref, ea_ref, w1e_ref, w2_ref, b2_ref, ge_ref,
                   bee_ref, msg_ref):
    pre = gs_ref[...] + gr_ref[...] + jnp.dot(
        ea_ref[...], w1e_ref[...], preferred_element_type=jnp.float32)
    h = jnp.maximum(pre, 0.0)
    m = jnp.dot(h, w2_ref[...], preferred_element_type=jnp.float32) + b2_ref[...]
    m = jnp.maximum(m, 0.0)
    msg_ref[...] = _ln(m, ge_ref[...], bee_ref[...])


def _edge_mlp(gs, gr, ea, w1e, w_e2, b_e2, g_e, be_e):
    blk = 2000
    grid = N_EDGES // blk
    return pl.pallas_call(
        _edge_mlp_body,
        grid=(grid,),
        in_specs=[
            pl.BlockSpec((blk, D), lambda i: (i, 0)),
            pl.BlockSpec((blk, D), lambda i: (i, 0)),
            pl.BlockSpec((blk, E_DIM), lambda i: (i, 0)),
            pl.BlockSpec((E_DIM, D), lambda i: (0, 0)),
            pl.BlockSpec((D, D), lambda i: (0, 0)),
            pl.BlockSpec((1, D), lambda i: (0, 0)),
            pl.BlockSpec((1, D), lambda i: (0, 0)),
            pl.BlockSpec((1, D), lambda i: (0, 0)),
        ],
        out_specs=pl.BlockSpec((blk, D), lambda i: (i, 0)),
        out_shape=jax.ShapeDtypeStruct((N_EDGES, D), jnp.float32),
    )(gs, gr, ea, w1e, w_e2, b_e2.reshape(1, D), g_e.reshape(1, D),
      be_e.reshape(1, D))


def _scatter_body(recv_ref, msg_ref, zero_ref, out_ref, idx_r, rows, acc):
    cid = lax.axis_index("c")
    sid = lax.axis_index("s")
    wid = sid * 2 + cid

    @pl.when(sid == 0)
    def _init():
        pltpu.sync_copy(zero_ref, acc)

    plsc.subcore_barrier()

    def body(i, carry):
        base = wid * EPW + i * CH
        pltpu.sync_copy(recv_ref.at[pl.ds(base, CH)], idx_r)
        pltpu.sync_copy(msg_ref.at[pl.ds(base, CH)], rows)
        pltpu.sync_copy(rows, acc.at[idx_r], add=True)
        return carry

    lax.fori_loop(0, NCH, body, 0)
    plsc.subcore_barrier()
    pltpu.sync_copy(acc.at[pl.ds(sid * RPS, RPS)],
                    out_ref.at[pl.ds(cid * N_NODES + sid * RPS, RPS)])

    @pl.when(sid == 15)
    def _tail():
        pltpu.sync_copy(acc.at[pl.ds(16 * RPS, N_NODES - 16 * RPS)],
                        out_ref.at[pl.ds(cid * N_NODES + 16 * RPS,
                                         N_NODES - 16 * RPS)])


def _scatter_agg(receivers, msg, zeros):
    mesh = plsc.VectorSubcoreMesh(core_axis_name="c", subcore_axis_name="s")
    return pl.kernel(
        _scatter_body,
        out_type=jax.ShapeDtypeStruct((2 * N_NODES, D), jnp.float32),
        mesh=mesh,
        scratch_types=[
            pltpu.VMEM((CH,), jnp.int32),
            pltpu.VMEM((CH, D), jnp.float32),
            pltpu.VMEM_SHARED((N_NODES, D), jnp.float32),
        ],
    )(receivers, msg, zeros)


def _node_mlp_body(x_ref, p0_ref, p1_ref, wn1a_ref, wn1b_ref, bn1_ref,
                   wn2_ref, bn2_ref, gn_ref, ben_ref, out_ref):
    x = x_ref[...]
    agg = p0_ref[...] + p1_ref[...]
    h = jnp.dot(x, wn1a_ref[...], preferred_element_type=jnp.float32)
    h = h + jnp.dot(agg, wn1b_ref[...], preferred_element_type=jnp.float32)
    h = jnp.maximum(h + bn1_ref[...], 0.0)
    upd = jnp.dot(h, wn2_ref[...], preferred_element_type=jnp.float32) + bn2_ref[...]
    out_ref[...] = _ln(x + upd, gn_ref[...], ben_ref[...])


def _node_mlp(x, partials, wn1a, wn1b, b_n1, w_n2, b_n2, g_n, be_n):
    blk = 2000
    grid = N_NODES // blk
    return pl.pallas_call(
        _node_mlp_body,
        grid=(grid,),
        in_specs=[
            pl.BlockSpec((blk, D), lambda i: (i, 0)),
            pl.BlockSpec((blk, D), lambda i: (i, 0)),
            pl.BlockSpec((blk, D), lambda i: (i + N_NODES // blk, 0)),
            pl.BlockSpec((D, D), lambda i: (0, 0)),
            pl.BlockSpec((D, D), lambda i: (0, 0)),
            pl.BlockSpec((1, D), lambda i: (0, 0)),
            pl.BlockSpec((D, D), lambda i: (0, 0)),
            pl.BlockSpec((1, D), lambda i: (0, 0)),
            pl.BlockSpec((1, D), lambda i: (0, 0)),
            pl.BlockSpec((1, D), lambda i: (0, 0)),
        ],
        out_specs=pl.BlockSpec((blk, D), lambda i: (i, 0)),
        out_shape=jax.ShapeDtypeStruct((N_NODES, D), jnp.float32),
    )(x, partials, partials, wn1a, wn1b, b_n1.reshape(1, D), w_n2,
      b_n2.reshape(1, D), g_n.reshape(1, D), be_n.reshape(1, D))


def kernel(x, edge_index, edge_attr, W_e1, b_e1, W_e2, b_e2, g_e, be_e,
           W_n1, b_n1, W_n2, b_n2, g_n, be_n):
    senders = edge_index[0].astype(jnp.int32)
    receivers = edge_index[1].astype(jnp.int32)

    w1s = W_e1[:D]
    w1r = W_e1[D:2 * D]
    w1e = W_e1[2 * D:]
    wn1a = W_n1[:D]
    wn1b = W_n1[D:]

    a, b = _premix(x, w1s, w1r, b_e1)
    gs, gr = _edge_gather(senders, receivers, a, b)
    msg = _edge_mlp(gs, gr, edge_attr, w1e, W_e2, b_e2, g_e, be_e)
    zeros = jnp.zeros((N_NODES, D), jnp.float32)
    partials = _scatter_agg(receivers, msg, zeros)
    return _node_mlp(x, partials, wn1a, wn1b, b_n1, W_n2, b_n2, g_n, be_n)


# speedup vs baseline: 4.3590x; 1.5861x over previous
import functools

import jax
import jax.numpy as jnp
from jax import lax
from jax.experimental import pallas as pl
from jax.experimental.pallas import tpu as pltpu
from jax.experimental.pallas import tpu_sc as plsc

N_NODES = 10000
N_EDGES = 320000
D = 128
E_DIM = 16

NW = 32
CH = 128
NCHT = N_EDGES // CH
RPS = 624

_EPS = 1e-5


def _ln(h, gamma, beta):
    mu = jnp.mean(h, axis=-1, keepdims=True)
    var = jnp.mean((h - mu) ** 2, axis=-1, keepdims=True)
    return (h - mu) * lax.rsqrt(var + _EPS) * gamma + beta


def _worker_chunks(wid):
    count = jnp.where(wid < 4, 79, 78)
    base = 78 * wid + jnp.minimum(wid, 4)
    return base, count


def _premix_body(x_ref, w1s_ref, w1r_ref, be1_ref, a_ref, b_ref):
    x = x_ref[...]
    a_ref[...] = jnp.dot(x, w1s_ref[...], preferred_element_type=jnp.float32)
    b_ref[...] = jnp.dot(x, w1r_ref[...], preferred_element_type=jnp.float32) + be1_ref[...]


def _premix(x, w1s, w1r, b_e1):
    blk = 2000
    grid = N_NODES // blk
    return pl.pallas_call(
        _premix_body,
        grid=(grid,),
        in_specs=[
            pl.BlockSpec((blk, D), lambda i: (i, 0)),
            pl.BlockSpec((D, D), lambda i: (0, 0)),
            pl.BlockSpec((D, D), lambda i: (0, 0)),
            pl.BlockSpec((1, D), lambda i: (0, 0)),
        ],
        out_specs=[
            pl.BlockSpec((blk, D), lambda i: (i, 0)),
            pl.BlockSpec((blk, D), lambda i: (i, 0)),
        ],
        out_shape=[
            jax.ShapeDtypeStruct((N_NODES, D), jnp.float32),
            jax.ShapeDtypeStruct((N_NODES, D), jnp.float32),
        ],
    )(x, w1s, w1r, b_e1.reshape(1, D))


def _gather_body(send_ref, recv_ref, a_ref, b_ref, g_ref,
                 idx_s0, idx_s1, idx_r0, idx_r1,
                 bufa0, bufa1, bufb0, bufb1, g0, g1,
                 sem_i0, sem_i1, sem_g0, sem_g1, sem_o0, sem_o1):
    cid = lax.axis_index("c")
    sid = lax.axis_index("s")
    wid = sid * 2 + cid
    base, count = _worker_chunks(wid)

    idx_s = (idx_s0, idx_s1)
    idx_r = (idx_r0, idx_r1)
    bufa = (bufa0, bufa1)
    bufb = (bufb0, bufb1)
    gbuf = (g0, g1)
    sem_i = (sem_i0, sem_i1)
    sem_g = (sem_g0, sem_g1)
    sem_o = (sem_o0, sem_o1)

    def idx_start(t, b):
        pltpu.async_copy(send_ref.at[base + t], idx_s[b], sem_i[b])
        pltpu.async_copy(recv_ref.at[base + t], idx_r[b], sem_i[b])

    def idx_wait(b):
        pltpu.make_async_copy(send_ref.at[0], idx_s[b], sem_i[b]).wait()
        pltpu.make_async_copy(recv_ref.at[0], idx_r[b], sem_i[b]).wait()

    def gat_start(b):
        pltpu.async_copy(a_ref.at[idx_s[b]], bufa[b], sem_g[b])
        pltpu.async_copy(b_ref.at[idx_r[b]], bufb[b], sem_g[b])

    def gat_wait(b):
        pltpu.make_async_copy(a_ref.at[idx_s[b]], bufa[b], sem_g[b]).wait()
        pltpu.make_async_copy(b_ref.at[idx_r[b]], bufb[b], sem_g[b]).wait()

    def wb_wait(b):
        pltpu.make_async_copy(gbuf[b], g_ref.at[pl.ds(0, CH)], sem_o[b]).wait()

    def add_wb(t, b):
        def row_group(rr, carry):
            for j in range(8):
                r = rr * 8 + j
                for k in range(8):
                    sl = pl.ds(k * 16, 16)
                    gbuf[b][r, sl] = bufa[b][r, sl] + bufb[b][r, sl]
            return carry

        lax.fori_loop(0, CH // 8, row_group, 0)
        pltpu.async_copy(gbuf[b], g_ref.at[pl.ds((base + t) * CH, CH)], sem_o[b])

    def pair(p, carry):
        for bb in range(2):
            t = p * 2 + bb
            b = bb

            @pl.when((t >= 2) & (t <= count + 1))
            def _wait_gather():
                gat_wait(b)

            @pl.when(t < count)
            def _prefetch_idx():
                idx_start(t, b)

            @pl.when((t >= 1) & (t <= count))
            def _start_gather():
                idx_wait(1 - b)
                gat_start(1 - b)

            @pl.when((t >= 2) & (t <= count + 1))
            def _add_writeback():
                @pl.when(t >= 4)
                def _wb_drain():
                    wb_wait(b)
                add_wb(t - 2, b)

        return carry

    lax.fori_loop(0, 41, pair, 0)
    wb_wait(0)
    wb_wait(1)


def _edge_gather(send2d, recv2d, a, b):
    mesh = plsc.VectorSubcoreMesh(core_axis_name="c", subcore_axis_name="s")
    return pl.kernel(
        _gather_body,
        out_type=jax.ShapeDtypeStruct((N_EDGES, D), jnp.float32),
        mesh=mesh,
        scratch_types=[
            pltpu.VMEM((CH,), jnp.int32), pltpu.VMEM((CH,), jnp.int32),
            pltpu.VMEM((CH,), jnp.int32), pltpu.VMEM((CH,), jnp.int32),
            pltpu.VMEM((CH, D), jnp.float32), pltpu.VMEM((CH, D), jnp.float32),
            pltpu.VMEM((CH, D), jnp.float32), pltpu.VMEM((CH, D), jnp.float32),
            pltpu.VMEM((CH, D), jnp.float32), pltpu.VMEM((CH, D), jnp.float32),
            pltpu.SemaphoreType.DMA, pltpu.SemaphoreType.DMA,
            pltpu.SemaphoreType.DMA, pltpu.SemaphoreType.DMA,
            pltpu.SemaphoreType.DMA, pltpu.SemaphoreType.DMA,
        ],
    )(send2d, recv2d, a, b)


def _edge_mlp_body(g_ref, ea_ref, w1e_ref, w2_ref, b2_ref, ge_ref,
                   bee_ref, msg_ref):
    pre = g_ref[...] + jnp.dot(
        ea_ref[...], w1e_ref[...], preferred_element_type=jnp.float32)
    h = jnp.maximum(pre, 0.0)
    m = jnp.dot(h, w2_ref[...], preferred_element_type=jnp.float32) + b2_ref[...]
    m = jnp.maximum(m, 0.0)
    msg_ref[...] = _ln(m, ge_ref[...], bee_ref[...])


def _edge_mlp(g, ea, w1e, w_e2, b_e2, g_e, be_e):
    blk = 2000
    grid = N_EDGES // blk
    return pl.pallas_call(
        _edge_mlp_body,
        grid=(grid,),
        in_specs=[
            pl.BlockSpec((blk, D), lambda i: (i, 0)),
            pl.BlockSpec((blk, E_DIM), lambda i: (i, 0)),
            pl.BlockSpec((E_DIM, D), lambda i: (0, 0)),
            pl.BlockSpec((D, D), lambda i: (0, 0)),
            pl.BlockSpec((1, D), lambda i: (0, 0)),
            pl.BlockSpec((1, D), lambda i: (0, 0)),
            pl.BlockSpec((1, D), lambda i: (0, 0)),
        ],
        out_specs=pl.BlockSpec((blk, D), lambda i: (i, 0)),
        out_shape=jax.ShapeDtypeStruct((N_EDGES, D), jnp.float32),
    )(g, ea, w1e, w_e2, b_e2.reshape(1, D), g_e.reshape(1, D),
      be_e.reshape(1, D))


def _scatter_body(recv_ref, msg_ref, zero_ref, out_ref,
                  idx0, idx1, rows0, rows1, acc,
                  sem_f0, sem_f1, sem_s0, sem_s1):
    cid = lax.axis_index("c")
    sid = lax.axis_index("s")
    wid = sid * 2 + cid
    base, count = _worker_chunks(wid)

    idx = (idx0, idx1)
    rows = (rows0, rows1)
    sem_f = (sem_f0, sem_f1)
    sem_s = (sem_s0, sem_s1)

    pltpu.sync_copy(zero_ref.at[pl.ds(sid * RPS, RPS)],
                    acc.at[pl.ds(sid * RPS, RPS)])

    @pl.when(sid == 15)
    def _zero_tail():
        pltpu.sync_copy(zero_ref.at[pl.ds(16 * RPS, N_NODES - 16 * RPS)],
                        acc.at[pl.ds(16 * RPS, N_NODES - 16 * RPS)])

    plsc.subcore_barrier()

    def fetch_start(t, b):
        pltpu.async_copy(recv_ref.at[base + t], idx[b], sem_f[b])
        pltpu.async_copy(msg_ref.at[pl.ds((base + t) * CH, CH)], rows[b], sem_f[b])

    def fetch_wait(b):
        pltpu.make_async_copy(recv_ref.at[0], idx[b], sem_f[b]).wait()
        pltpu.make_async_copy(msg_ref.at[pl.ds(0, CH)], rows[b], sem_f[b]).wait()

    def scat_start(b):
        pltpu.async_copy(rows[b], acc.at[idx[b]], sem_s[b], add=True)

    def scat_wait(b):
        pltpu.make_async_copy(rows[b], acc.at[idx[b]], sem_s[b]).wait()

    def pair(p, carry):
        for bb in range(2):
            t = p * 2 + bb
            b = bb

            @pl.when(t < count)
            def _fetch():
                @pl.when(t >= 2)
                def _scat_drain():
                    scat_wait(b)
                fetch_start(t, b)

            @pl.when((t >= 1) & (t <= count))
            def _scatter():
                fetch_wait(1 - b)
                scat_start(1 - b)

        return carry

    lax.fori_loop(0, 41, pair, 0)
    scat_wait(0)
    scat_wait(1)

    plsc.subcore_barrier()
    pltpu.sync_copy(acc.at[pl.ds(sid * RPS, RPS)],
                    out_ref.at[pl.ds(cid * N_NODES + sid * RPS, RPS)])

    @pl.when(sid == 15)
    def _tail():
        pltpu.sync_copy(acc.at[pl.ds(16 * RPS, N_NODES - 16 * RPS)],
                        out_ref.at[pl.ds(cid * N_NODES + 16 * RPS,
                                         N_NODES - 16 * RPS)])


def _scatter_agg(recv2d, msg, zeros):
    mesh = plsc.VectorSubcoreMesh(core_axis_name="c", subcore_axis_name="s")
    return pl.kernel(
        _scatter_body,
        out_type=jax.ShapeDtypeStruct((2 * N_NODES, D), jnp.float32),
        mesh=mesh,
        scratch_types=[
            pltpu.VMEM((CH,), jnp.int32), pltpu.VMEM((CH,), jnp.int32),
            pltpu.VMEM((CH, D), jnp.float32), pltpu.VMEM((CH, D), jnp.float32),
            pltpu.VMEM_SHARED((N_NODES, D), jnp.float32),
            pltpu.SemaphoreType.DMA, pltpu.SemaphoreType.DMA,
            pltpu.SemaphoreType.DMA, pltpu.SemaphoreType.DMA,
        ],
    )(recv2d, msg, zeros)


def _node_mlp_body(x_ref, p0_ref, p1_ref, wn1a_ref, wn1b_ref, bn1_ref,
                   wn2_ref, bn2_ref, gn_ref, ben_ref, out_ref):
    x = x_ref[...]
    agg = p0_ref[...] + p1_ref[...]
    h = jnp.dot(x, wn1a_ref[...], preferred_element_type=jnp.float32)
    h = h + jnp.dot(agg, wn1b_ref[...], preferred_element_type=jnp.float32)
    h = jnp.maximum(h + bn1_ref[...], 0.0)
    upd = jnp.dot(h, wn2_ref[...], preferred_element_type=jnp.float32) + bn2_ref[...]
    out_ref[...] = _ln(x + upd, gn_ref[...], ben_ref[...])


def _node_mlp(x, partials, wn1a, wn1b, b_n1, w_n2, b_n2, g_n, be_n):
    blk = 2000
    grid = N_NODES // blk
    return pl.pallas_call(
        _node_mlp_body,
        grid=(grid,),
        in_specs=[
            pl.BlockSpec((blk, D), lambda i: (i, 0)),
            pl.BlockSpec((blk, D), lambda i: (i, 0)),
            pl.BlockSpec((blk, D), lambda i: (i + N_NODES // blk, 0)),
            pl.BlockSpec((D, D), lambda i: (0, 0)),
            pl.BlockSpec((D, D), lambda i: (0, 0)),
            pl.BlockSpec((1, D), lambda i: (0, 0)),
            pl.BlockSpec((D, D), lambda i: (0, 0)),
            pl.BlockSpec((1, D), lambda i: (0, 0)),
            pl.BlockSpec((1, D), lambda i: (0, 0)),
            pl.BlockSpec((1, D), lambda i: (0, 0)),
        ],
        out_specs=pl.BlockSpec((blk, D), lambda i: (i, 0)),
        out_shape=jax.ShapeDtypeStruct((N_NODES, D), jnp.float32),
    )(x, partials, partials, wn1a, wn1b, b_n1.reshape(1, D), w_n2,
      b_n2.reshape(1, D), g_n.reshape(1, D), be_n.reshape(1, D))


def kernel(x, edge_index, edge_attr, W_e1, b_e1, W_e2, b_e2, g_e, be_e,
           W_n1, b_n1, W_n2, b_n2, g_n, be_n):
    send2d = edge_index[0].astype(jnp.int32).reshape(NCHT, CH)
    recv2d = edge_index[1].astype(jnp.int32).reshape(NCHT, CH)

    w1s = W_e1[:D]
    w1r = W_e1[D:2 * D]
    w1e = W_e1[2 * D:]
    wn1a = W_n1[:D]
    wn1b = W_n1[D:]

    a, b = _premix(x, w1s, w1r, b_e1)
    g = _edge_gather(send2d, recv2d, a, b)
    msg = _edge_mlp(g, edge_attr, w1e, W_e2, b_e2, g_e, be_e)
    zeros = jnp.zeros((N_NODES, D), jnp.float32)
    partials = _scatter_agg(recv2d, msg, zeros)
    return _node_mlp(x, partials, wn1a, wn1b, b_n1, W_n2, b_n2, g_n, be_n)


# speedup vs baseline: 4.6343x; 1.0631x over previous
import functools

import jax
import jax.numpy as jnp
from jax import lax
from jax.experimental import pallas as pl
from jax.experimental.pallas import tpu as pltpu
from jax.experimental.pallas import tpu_sc as plsc

N_NODES = 10000
N_EDGES = 320000
D = 128
E_DIM = 16

NW = 32
CH = 128
NCHT = N_EDGES // CH
SPLIT = 2
CPS = NCHT // SPLIT
RPS = 624

_EPS = 1e-5


def _ln(h, gamma, beta):
    mu = jnp.mean(h, axis=-1, keepdims=True)
    var = jnp.mean((h - mu) ** 2, axis=-1, keepdims=True)
    return (h - mu) * lax.rsqrt(var + _EPS) * gamma + beta


def _worker_chunks(wid, total):
    q, r = total // NW, total % NW
    count = q + (wid < r).astype(jnp.int32)
    base = q * wid + jnp.minimum(wid, r)
    return base, count


def _npairs(total):
    q, r = total // NW, total % NW
    return (q + (1 if r else 0) + 2 + 1) // 2 + 1


def _premix_body(x_ref, w1s_ref, w1r_ref, be1_ref, a_ref, b_ref):
    x = x_ref[...]
    a_ref[...] = jnp.dot(x, w1s_ref[...], preferred_element_type=jnp.float32)
    b_ref[...] = jnp.dot(x, w1r_ref[...], preferred_element_type=jnp.float32) + be1_ref[...]


def _premix(x, w1s, w1r, b_e1):
    blk = 2000
    grid = N_NODES // blk
    return pl.pallas_call(
        _premix_body,
        grid=(grid,),
        in_specs=[
            pl.BlockSpec((blk, D), lambda i: (i, 0)),
            pl.BlockSpec((D, D), lambda i: (0, 0)),
            pl.BlockSpec((D, D), lambda i: (0, 0)),
            pl.BlockSpec((1, D), lambda i: (0, 0)),
        ],
        out_specs=[
            pl.BlockSpec((blk, D), lambda i: (i, 0)),
            pl.BlockSpec((blk, D), lambda i: (i, 0)),
        ],
        out_shape=[
            jax.ShapeDtypeStruct((N_NODES, D), jnp.float32),
            jax.ShapeDtypeStruct((N_NODES, D), jnp.float32),
        ],
    )(x, w1s, w1r, b_e1.reshape(1, D))


def _gather_body(chunk0, nchunks, send_ref, recv_ref, a_ref, b_ref, g_ref,
                 idx_s0, idx_s1, idx_r0, idx_r1,
                 bufa0, bufa1, bufb0, bufb1, g0, g1,
                 sem_i0, sem_i1, sem_g0, sem_g1, sem_o0, sem_o1):
    cid = lax.axis_index("c")
    sid = lax.axis_index("s")
    wid = sid * 2 + cid
    base, count = _worker_chunks(wid, nchunks)

    idx_s = (idx_s0, idx_s1)
    idx_r = (idx_r0, idx_r1)
    bufa = (bufa0, bufa1)
    bufb = (bufb0, bufb1)
    gbuf = (g0, g1)
    sem_i = (sem_i0, sem_i1)
    sem_g = (sem_g0, sem_g1)
    sem_o = (sem_o0, sem_o1)

    def idx_start(t, b):
        pltpu.async_copy(send_ref.at[chunk0 + base + t], idx_s[b], sem_i[b])
        pltpu.async_copy(recv_ref.at[chunk0 + base + t], idx_r[b], sem_i[b])

    def idx_wait(b):
        pltpu.make_async_copy(send_ref.at[0], idx_s[b], sem_i[b]).wait()
        pltpu.make_async_copy(recv_ref.at[0], idx_r[b], sem_i[b]).wait()

    def gat_start(b):
        pltpu.async_copy(a_ref.at[idx_s[b]], bufa[b], sem_g[b])
        pltpu.async_copy(b_ref.at[idx_r[b]], bufb[b], sem_g[b])

    def gat_wait(b):
        pltpu.make_async_copy(a_ref.at[idx_s[b]], bufa[b], sem_g[b]).wait()
        pltpu.make_async_copy(b_ref.at[idx_r[b]], bufb[b], sem_g[b]).wait()

    def wb_wait(b):
        pltpu.make_async_copy(gbuf[b], g_ref.at[pl.ds(0, CH)], sem_o[b]).wait()

    def add_wb(t, b):
        def row_group(rr, carry):
            for j in range(8):
                r = rr * 8 + j
                for k in range(8):
                    sl = pl.ds(k * 16, 16)
                    gbuf[b][r, sl] = bufa[b][r, sl] + bufb[b][r, sl]
            return carry

        lax.fori_loop(0, CH // 8, row_group, 0)
        pltpu.async_copy(gbuf[b], g_ref.at[pl.ds((base + t) * CH, CH)], sem_o[b])

    def pair(p, carry):
        for bb in range(2):
            t = p * 2 + bb
            b = bb

            @pl.when((t >= 2) & (t <= count + 1))
            def _wait_gather():
                gat_wait(b)

            @pl.when(t < count)
            def _prefetch_idx():
                idx_start(t, b)

            @pl.when((t >= 1) & (t <= count))
            def _start_gather():
                idx_wait(1 - b)
                gat_start(1 - b)

            @pl.when((t >= 2) & (t <= count + 1))
            def _add_writeback():
                @pl.when(t >= 4)
                def _wb_drain():
                    wb_wait(b)
                add_wb(t - 2, b)

        return carry

    lax.fori_loop(0, _npairs(nchunks), pair, 0)
    wb_wait(0)
    wb_wait(1)


def _edge_gather(chunk0, nchunks, send2d, recv2d, a, b):
    mesh = plsc.VectorSubcoreMesh(core_axis_name="c", subcore_axis_name="s")
    return pl.kernel(
        functools.partial(_gather_body, chunk0, nchunks),
        out_type=jax.ShapeDtypeStruct((nchunks * CH, D), jnp.float32),
        mesh=mesh,
        scratch_types=[
            pltpu.VMEM((CH,), jnp.int32), pltpu.VMEM((CH,), jnp.int32),
            pltpu.VMEM((CH,), jnp.int32), pltpu.VMEM((CH,), jnp.int32),
            pltpu.VMEM((CH, D), jnp.float32), pltpu.VMEM((CH, D), jnp.float32),
            pltpu.VMEM((CH, D), jnp.float32), pltpu.VMEM((CH, D), jnp.float32),
            pltpu.VMEM((CH, D), jnp.float32), pltpu.VMEM((CH, D), jnp.float32),
            pltpu.SemaphoreType.DMA, pltpu.SemaphoreType.DMA,
            pltpu.SemaphoreType.DMA, pltpu.SemaphoreType.DMA,
            pltpu.SemaphoreType.DMA, pltpu.SemaphoreType.DMA,
        ],
    )(send2d, recv2d, a, b)


def _edge_mlp_body(g_ref, ea_ref, w1e_ref, w2_ref, b2_ref, ge_ref,
                   bee_ref, msg_ref):
    pre = g_ref[...] + jnp.dot(
        ea_ref[...], w1e_ref[...], preferred_element_type=jnp.float32)
    h = jnp.maximum(pre, 0.0)
    m = jnp.dot(h, w2_ref[...], preferred_element_type=jnp.float32) + b2_ref[...]
    m = jnp.maximum(m, 0.0)
    msg_ref[...] = _ln(m, ge_ref[...], bee_ref[...])


def _edge_mlp(g, ea, w1e, w_e2, b_e2, g_e, be_e):
    blk = 2000
    nrows = g.shape[0]
    grid = nrows // blk
    return pl.pallas_call(
        _edge_mlp_body,
        grid=(grid,),
        in_specs=[
            pl.BlockSpec((blk, D), lambda i: (i, 0)),
            pl.BlockSpec((blk, E_DIM), lambda i: (i, 0)),
            pl.BlockSpec((E_DIM, D), lambda i: (0, 0)),
            pl.BlockSpec((D, D), lambda i: (0, 0)),
            pl.BlockSpec((1, D), lambda i: (0, 0)),
            pl.BlockSpec((1, D), lambda i: (0, 0)),
            pl.BlockSpec((1, D), lambda i: (0, 0)),
        ],
        out_specs=pl.BlockSpec((blk, D), lambda i: (i, 0)),
        out_shape=jax.ShapeDtypeStruct((nrows, D), jnp.float32),
    )(g, ea, w1e, w_e2, b_e2.reshape(1, D), g_e.reshape(1, D),
      be_e.reshape(1, D))


def _scatter_body(chunk0, nchunks, recv_ref, msg_ref, init_ref, out_ref,
                  idx0, idx1, rows0, rows1, acc,
                  sem_f0, sem_f1, sem_s0, sem_s1):
    cid = lax.axis_index("c")
    sid = lax.axis_index("s")
    wid = sid * 2 + cid
    base, count = _worker_chunks(wid, nchunks)

    idx = (idx0, idx1)
    rows = (rows0, rows1)
    sem_f = (sem_f0, sem_f1)
    sem_s = (sem_s0, sem_s1)

    init_off = cid * (init_ref.shape[0] // 2) if init_ref.shape[0] == 2 * N_NODES else 0

    pltpu.sync_copy(init_ref.at[pl.ds(init_off + sid * RPS, RPS)],
                    acc.at[pl.ds(sid * RPS, RPS)])

    @pl.when(sid == 15)
    def _init_tail():
        pltpu.sync_copy(init_ref.at[pl.ds(init_off + 16 * RPS, N_NODES - 16 * RPS)],
                        acc.at[pl.ds(16 * RPS, N_NODES - 16 * RPS)])

    plsc.subcore_barrier()

    def fetch_start(t, b):
        pltpu.async_copy(recv_ref.at[chunk0 + base + t], idx[b], sem_f[b])
        pltpu.async_copy(msg_ref.at[pl.ds((base + t) * CH, CH)], rows[b], sem_f[b])

    def fetch_wait(b):
        pltpu.make_async_copy(recv_ref.at[0], idx[b], sem_f[b]).wait()
        pltpu.make_async_copy(msg_ref.at[pl.ds(0, CH)], rows[b], sem_f[b]).wait()

    def scat_start(b):
        pltpu.async_copy(rows[b], acc.at[idx[b]], sem_s[b], add=True)

    def scat_wait(b):
        pltpu.make_async_copy(rows[b], acc.at[idx[b]], sem_s[b]).wait()

    def pair(p, carry):
        for bb in range(2):
            t = p * 2 + bb
            b = bb

            @pl.when(t < count)
            def _fetch():
                @pl.when(t >= 2)
                def _scat_drain():
                    scat_wait(b)
                fetch_start(t, b)

            @pl.when((t >= 1) & (t <= count))
            def _scatter():
                fetch_wait(1 - b)
                scat_start(1 - b)

        return carry

    lax.fori_loop(0, _npairs(nchunks), pair, 0)
    scat_wait(0)
    scat_wait(1)

    plsc.subcore_barrier()
    pltpu.sync_copy(acc.at[pl.ds(sid * RPS, RPS)],
                    out_ref.at[pl.ds(cid * N_NODES + sid * RPS, RPS)])

    @pl.when(sid == 15)
    def _out_tail():
        pltpu.sync_copy(acc.at[pl.ds(16 * RPS, N_NODES - 16 * RPS)],
                        out_ref.at[pl.ds(cid * N_NODES + 16 * RPS,
                                         N_NODES - 16 * RPS)])


def _scatter_agg(chunk0, nchunks, recv2d, msg, init):
    mesh = plsc.VectorSubcoreMesh(core_axis_name="c", subcore_axis_name="s")
    return pl.kernel(
        functools.partial(_scatter_body, chunk0, nchunks),
        out_type=jax.ShapeDtypeStruct((2 * N_NODES, D), jnp.float32),
        mesh=mesh,
        scratch_types=[
            pltpu.VMEM((CH,), jnp.int32), pltpu.VMEM((CH,), jnp.int32),
            pltpu.VMEM((CH, D), jnp.float32), pltpu.VMEM((CH, D), jnp.float32),
            pltpu.VMEM_SHARED((N_NODES, D), jnp.float32),
            pltpu.SemaphoreType.DMA, pltpu.SemaphoreType.DMA,
            pltpu.SemaphoreType.DMA, pltpu.SemaphoreType.DMA,
        ],
    )(recv2d, msg, init)


def _node_mlp_body(x_ref, p0_ref, p1_ref, wn1a_ref, wn1b_ref, bn1_ref,
                   wn2_ref, bn2_ref, gn_ref, ben_ref, out_ref):
    x = x_ref[...]
    agg = p0_ref[...] + p1_ref[...]
    h = jnp.dot(x, wn1a_ref[...], preferred_element_type=jnp.float32)
    h = h + jnp.dot(agg, wn1b_ref[...], preferred_element_type=jnp.float32)
    h = jnp.maximum(h + bn1_ref[...], 0.0)
    upd = jnp.dot(h, wn2_ref[...], preferred_element_type=jnp.float32) + bn2_ref[...]
    out_ref[...] = _ln(x + upd, gn_ref[...], ben_ref[...])


def _node_mlp(x, partials, wn1a, wn1b, b_n1, w_n2, b_n2, g_n, be_n):
    blk = 2000
    grid = N_NODES // blk
    return pl.pallas_call(
        _node_mlp_body,
        grid=(grid,),
        in_specs=[
            pl.BlockSpec((blk, D), lambda i: (i, 0)),
            pl.BlockSpec((blk, D), lambda i: (i, 0)),
            pl.BlockSpec((blk, D), lambda i: (i + N_NODES // blk, 0)),
            pl.BlockSpec((D, D), lambda i: (0, 0)),
            pl.BlockSpec((D, D), lambda i: (0, 0)),
            pl.BlockSpec((1, D), lambda i: (0, 0)),
            pl.BlockSpec((D, D), lambda i: (0, 0)),
            pl.BlockSpec((1, D), lambda i: (0, 0)),
            pl.BlockSpec((1, D), lambda i: (0, 0)),
            pl.BlockSpec((1, D), lambda i: (0, 0)),
        ],
        out_specs=pl.BlockSpec((blk, D), lambda i: (i, 0)),
        out_shape=jax.ShapeDtypeStruct((N_NODES, D), jnp.float32),
    )(x, partials, partials, wn1a, wn1b, b_n1.reshape(1, D), w_n2,
      b_n2.reshape(1, D), g_n.reshape(1, D), be_n.reshape(1, D))


def kernel(x, edge_index, edge_attr, W_e1, b_e1, W_e2, b_e2, g_e, be_e,
           W_n1, b_n1, W_n2, b_n2, g_n, be_n):
    send2d = edge_index[0].astype(jnp.int32).reshape(NCHT, CH)
    recv2d = edge_index[1].astype(jnp.int32).reshape(NCHT, CH)

    w1s = W_e1[:D]
    w1r = W_e1[D:2 * D]
    w1e = W_e1[2 * D:]
    wn1a = W_n1[:D]
    wn1b = W_n1[D:]

    a, b = _premix(x, w1s, w1r, b_e1)

    gs = [_edge_gather(s * CPS, CPS, send2d, recv2d, a, b)
          for s in range(SPLIT)]
    msgs = [_edge_mlp(g, lax.dynamic_slice_in_dim(edge_attr, s * CPS * CH, CPS * CH),
                      w1e, W_e2, b_e2, g_e, be_e)
            for s, g in enumerate(gs)]

    partials = jnp.zeros((N_NODES, D), jnp.float32)
    for s, m in enumerate(msgs):
        partials = _scatter_agg(s * CPS, CPS, recv2d, m, partials)

    return _node_mlp(x, partials, wn1a, wn1b, b_n1, W_n2, b_n2, g_n, be_n)
